# baseline (speedup 1.0000x reference)
import functools

import jax
import jax.numpy as jnp
from jax.experimental import pallas as pl
from jax.experimental.pallas import tpu as pltpu

_VMEM_LIMIT = 48 * 1024 * 1024


def _ln_rowsum(x, eps):
    d = x.shape[-1]
    s1 = jnp.sum(x, axis=-1, keepdims=True)
    s2 = jnp.sum(x * x, axis=-1, keepdims=True)
    mu = s1 * (1.0 / d)
    var = s2 * (1.0 / d) - mu * mu
    r = jax.lax.rsqrt(var + eps)
    big = jnp.sum(x * r, axis=1)
    t = jnp.sum(mu * r, axis=1)
    return big - t


def _prenorm_pool_kernel(x1_ref, x2_ref, g_ref, b_ref, w_ref, wb_ref, o_ref,
                         *, eps, seq):
    s = _ln_rowsum(x1_ref[...], eps) + _ln_rowsum(x2_ref[...], eps)
    pooled = s * (1.0 / seq) * g_ref[...] + b_ref[...]
    y = jnp.dot(pooled, w_ref[...], preferred_element_type=jnp.float32)
    o_ref[0] = y + wb_ref[...]


def kernel(ctx, g, b, w, wb):
    bsz, seq, din = ctx.shape
    dout = w.shape[-1]
    tb = 4
    grid = (bsz // tb,)

    out = pl.pallas_call(
        functools.partial(_prenorm_pool_kernel, eps=1e-5, seq=seq),
        out_shape=jax.ShapeDtypeStruct((bsz // tb, tb, dout), jnp.float32),
        grid=grid,
        in_specs=[
            pl.BlockSpec((tb, seq // 2, din), lambda i: (i, 0, 0)),
            pl.BlockSpec((tb, seq // 2, din), lambda i: (i, 1, 0)),
            pl.BlockSpec((1, din), lambda i: (0, 0)),
            pl.BlockSpec((1, din), lambda i: (0, 0)),
            pl.BlockSpec((din, dout), lambda i: (0, 0)),
            pl.BlockSpec((1, dout), lambda i: (0, 0)),
        ],
        out_specs=pl.BlockSpec((1, tb, dout), lambda i: (i, 0, 0)),
        compiler_params=pltpu.CompilerParams(
            dimension_semantics=("parallel",),
            vmem_limit_bytes=_VMEM_LIMIT),
    )(ctx, ctx, g.reshape(1, din), b.reshape(1, din), w, wb.reshape(1, dout))
    return out.reshape(bsz, 1, dout)


# speedup vs baseline: 1.2907x; 1.2907x over previous
import functools

import jax
import jax.numpy as jnp
from jax.experimental import pallas as pl
from jax.experimental.pallas import tpu as pltpu

_VMEM_LIMIT = 56 * 1024 * 1024


def _prenorm_pool_kernel(ctx_ref, g_ref, b_ref, w_ref, wb_ref, o_ref, *, eps,
                         seq):
    x = ctx_ref[...]
    tb, _, d = x.shape
    s1 = jnp.sum(x, axis=-1, keepdims=True)
    s2 = jnp.sum(x * x, axis=-1, keepdims=True)
    mu = s1 * (1.0 / d)
    var = s2 * (1.0 / d) - mu * mu
    r = jax.lax.rsqrt(var + eps)

    x2 = x.reshape(tb * seq, d)
    r_row = r.reshape(1, tb * seq)
    blk = jax.lax.broadcasted_iota(jnp.int32, (tb, tb * seq), 1) // seq
    row = jax.lax.broadcasted_iota(jnp.int32, (tb, tb * seq), 0)
    rm = jnp.where(blk == row, r_row, 0.0)
    big = jnp.dot(rm, x2, preferred_element_type=jnp.float32)
    t = jnp.sum(mu * r, axis=1)
    s = big - t

    pooled = s * (1.0 / seq) * g_ref[...] + b_ref[...]
    y = jnp.dot(pooled, w_ref[...], preferred_element_type=jnp.float32)
    o_ref[0] = y + wb_ref[...]


def kernel(ctx, g, b, w, wb):
    bsz, seq, din = ctx.shape
    dout = w.shape[-1]
    tb = 8
    grid = (bsz // tb,)

    out = pl.pallas_call(
        functools.partial(_prenorm_pool_kernel, eps=1e-5, seq=seq),
        out_shape=jax.ShapeDtypeStruct((bsz // tb, tb, dout), jnp.float32),
        grid=grid,
        in_specs=[
            pl.BlockSpec((tb, seq, din), lambda i: (i, 0, 0)),
            pl.BlockSpec((1, din), lambda i: (0, 0)),
            pl.BlockSpec((1, din), lambda i: (0, 0)),
            pl.BlockSpec((din, dout), lambda i: (0, 0)),
            pl.BlockSpec((1, dout), lambda i: (0, 0)),
        ],
        out_specs=pl.BlockSpec((1, tb, dout), lambda i: (i, 0, 0)),
        compiler_params=pltpu.CompilerParams(
            dimension_semantics=("parallel",),
            vmem_limit_bytes=_VMEM_LIMIT),
    )(ctx, g.reshape(1, din), b.reshape(1, din), w, wb.reshape(1, dout))
    return out.reshape(bsz, 1, dout)
